# baseline (speedup 1.0000x reference)
import jax
import jax.numpy as jnp
from jax.experimental import pallas as pl


def _add_kernel(x_ref, pos_ref, o_ref):
    o_ref[...] = x_ref[...] + pos_ref[...][None, :, :]


def kernel(patch_embeddings, pos_table):
    batch, seq, dim = patch_embeddings.shape
    pos = pos_table[:seq]
    grid = (batch,)
    return pl.pallas_call(
        _add_kernel,
        grid=grid,
        in_specs=[
            pl.BlockSpec((1, seq, dim), lambda b: (b, 0, 0)),
            pl.BlockSpec((seq, dim), lambda b: (0, 0)),
        ],
        out_specs=pl.BlockSpec((1, seq, dim), lambda b: (b, 0, 0)),
        out_shape=jax.ShapeDtypeStruct((batch, seq, dim), patch_embeddings.dtype),
    )(patch_embeddings, pos)


# speedup vs baseline: 1.2033x; 1.2033x over previous
import jax
import jax.numpy as jnp
from jax.experimental import pallas as pl


_BB = 8


def _add_kernel(x_ref, pos_ref, o_ref):
    o_ref[...] = x_ref[...] + pos_ref[...][None, :, :]


def kernel(patch_embeddings, pos_table):
    batch, seq, dim = patch_embeddings.shape
    pos = pos_table[:seq]
    grid = (batch // _BB,)
    return pl.pallas_call(
        _add_kernel,
        grid=grid,
        in_specs=[
            pl.BlockSpec((_BB, seq, dim), lambda b: (b, 0, 0)),
            pl.BlockSpec((seq, dim), lambda b: (0, 0)),
        ],
        out_specs=pl.BlockSpec((_BB, seq, dim), lambda b: (b, 0, 0)),
        out_shape=jax.ShapeDtypeStruct((batch, seq, dim), patch_embeddings.dtype),
    )(patch_embeddings, pos)
